# Initial kernel scaffold; baseline (speedup 1.0000x reference)
import jax
import jax.numpy as jnp
from jax.experimental import pallas as pl


def kernel(x, edge_index, edge_attr, batch, expander_edge_index, expander_node_mask, num_nodes, emb, Wedge, bedge, eps_main, W1, b1, W2, b2, bn_g, bn_b, eps_r, rW1, rb1, rW2, rb2, rbn_g, rbn_b):
    raise NotImplementedError("write your pallas kernel here")


# speedup vs baseline: 1.9161x; 1.9161x over previous
import functools

import jax
import jax.numpy as jnp
from jax import lax
from jax.experimental import pallas as pl
from jax.experimental.pallas import tpu as pltpu
from jax.experimental.pallas import tpu_sc as plsc

NN = 15000
DD = 128
DE = 16
HH = 256
NC = 2
NS = 16
ROWS_PER_TILE = 944
LAST_ROWS = 848
NPAD = 15 * ROWS_PER_TILE + LAST_ROWS
KE = 40


def _sc_body(has_e, nchunk, epw, *refs):
    if has_e:
        (h_hbm, src_hbm, dst_hbm, e_hbm, z_hbm, out_hbm,
         shared, idx_s, idx_d, rows, erows, sem) = refs
    else:
        (h_hbm, src_hbm, dst_hbm, z_hbm, out_hbm,
         shared, idx_s, idx_d, rows, sem) = refs
    c = lax.axis_index("c")
    s = lax.axis_index("s")
    wid = c * NS + s

    off_rows = pl.multiple_of(s * ROWS_PER_TILE, 8)

    @pl.when(s < NS - 1)
    def _():
        pltpu.sync_copy(z_hbm, shared.at[pl.ds(off_rows, ROWS_PER_TILE)])

    @pl.when(s == NS - 1)
    def _():
        pltpu.sync_copy(z_hbm.at[pl.ds(0, LAST_ROWS)],
                        shared.at[pl.ds(off_rows, LAST_ROWS)])

    plsc.subcore_barrier()

    base = wid * epw

    def chunk(i, carry):
        off = pl.multiple_of(base + i * KE, 8)
        pltpu.sync_copy(src_hbm.at[pl.ds(off, KE)], idx_s)
        pltpu.sync_copy(dst_hbm.at[pl.ds(off, KE)], idx_d)
        pltpu.async_copy(h_hbm.at[idx_s], rows, sem).wait()
        if has_e:
            pltpu.sync_copy(e_hbm.at[pl.ds(off, KE)], erows)

            def rowloop(r, cc):
                for j in range(DD // 16):
                    sl = pl.ds(j * 16, 16)
                    rows[r, sl] = jnp.maximum(rows[r, sl] + erows[r, sl], 0.0)
                return cc

            lax.fori_loop(0, KE, rowloop, 0)
        pltpu.sync_copy(rows, shared.at[idx_d], add=True)
        return carry

    lax.fori_loop(0, nchunk, chunk, 0)
    plsc.subcore_barrier()

    @pl.when(s < NS - 1)
    def _():
        pltpu.sync_copy(shared.at[pl.ds(off_rows, ROWS_PER_TILE)],
                        out_hbm.at[c, pl.ds(off_rows, ROWS_PER_TILE)])

    @pl.when(s == NS - 1)
    def _():
        pltpu.sync_copy(shared.at[pl.ds(off_rows, LAST_ROWS)],
                        out_hbm.at[c, pl.ds(off_rows, LAST_ROWS)])


@functools.lru_cache(maxsize=None)
def _make_sc_pass(num_edges, has_e):
    epw = num_edges // (NC * NS)
    assert epw * NC * NS == num_edges and epw % KE == 0
    nchunk = epw // KE
    scratch = [
        pltpu.VMEM_SHARED((NPAD, DD), jnp.float32),
        pltpu.VMEM((KE,), jnp.int32),
        pltpu.VMEM((KE,), jnp.int32),
        pltpu.VMEM((KE, DD), jnp.float32),
    ]
    if has_e:
        scratch.append(pltpu.VMEM((KE, DD), jnp.float32))
    scratch.append(pltpu.SemaphoreType.DMA)
    return pl.kernel(
        functools.partial(_sc_body, has_e, nchunk, epw),
        out_type=jax.ShapeDtypeStruct((NC, NPAD, DD), jnp.float32),
        mesh=plsc.VectorSubcoreMesh(core_axis_name="c", subcore_axis_name="s"),
        scratch_types=scratch,
    )


NBLK = 1000


def _h0_body(m_ref, e_ref, o_ref):
    o_ref[...] = m_ref[...] * e_ref[...]


def _h0(mask2d, emb):
    return pl.pallas_call(
        _h0_body,
        grid=(NN // NBLK,),
        in_specs=[pl.BlockSpec((NBLK, 1), lambda i: (i, 0)),
                  pl.BlockSpec((1, DD), lambda i: (0, 0))],
        out_specs=pl.BlockSpec((NBLK, DD), lambda i: (i, 0)),
        out_shape=jax.ShapeDtypeStruct((NN, DD), jnp.float32),
    )(mask2d, emb)


def _edge_matmul_body(a_ref, w_ref, b_ref, o_ref):
    o_ref[...] = jnp.dot(a_ref[...], w_ref[...],
                         preferred_element_type=jnp.float32) + b_ref[...]


def _edge_matmul(edge_attr, W, b):
    E = edge_attr.shape[0]
    blk = 4000
    return pl.pallas_call(
        _edge_matmul_body,
        grid=(E // blk,),
        in_specs=[pl.BlockSpec((blk, DE), lambda i: (i, 0)),
                  pl.BlockSpec((DE, DD), lambda i: (0, 0)),
                  pl.BlockSpec((1, DD), lambda i: (0, 0))],
        out_specs=pl.BlockSpec((blk, DD), lambda i: (i, 0)),
        out_shape=jax.ShapeDtypeStruct((E, DD), jnp.float32),
    )(edge_attr, W, b.reshape(1, DD))


def _mlp_body(relu_out, emit_masked, s_ref, h_ref, p_ref0, p_ref1,
              w1_ref, b1_ref, w2_ref, b2_ref, *rest):
    if emit_masked:
        m_ref, o_ref, om_ref = rest
    else:
        (o_ref,) = rest
    z = s_ref[0, 0] * h_ref[...] + p_ref0[0] + p_ref1[0]
    a = jnp.maximum(
        jnp.dot(z, w1_ref[...], preferred_element_type=jnp.float32)
        + b1_ref[...], 0.0)
    y = jnp.dot(a, w2_ref[...], preferred_element_type=jnp.float32) + b2_ref[...]
    if relu_out:
        y = jnp.maximum(y, 0.0)
    o_ref[...] = y
    if emit_masked:
        om_ref[...] = y * m_ref[...]


def _node_mlp(h, parts, eps, W1, b1, W2f, b2f, relu_out, mask2d=None):
    emit_masked = mask2d is not None
    scale = (1.0 + eps).reshape(1, 1)
    in_specs = [
        pl.BlockSpec(memory_space=pltpu.SMEM),
        pl.BlockSpec((NBLK, DD), lambda i: (i, 0)),
        pl.BlockSpec((1, NBLK, DD), lambda i: (0, i, 0)),
        pl.BlockSpec((1, NBLK, DD), lambda i: (1, i, 0)),
        pl.BlockSpec((DD, HH), lambda i: (0, 0)),
        pl.BlockSpec((1, HH), lambda i: (0, 0)),
        pl.BlockSpec((HH, DD), lambda i: (0, 0)),
        pl.BlockSpec((1, DD), lambda i: (0, 0)),
    ]
    args = [scale, h, parts, parts, W1, b1.reshape(1, HH), W2f,
            b2f.reshape(1, DD)]
    if emit_masked:
        in_specs.append(pl.BlockSpec((NBLK, 1), lambda i: (i, 0)))
        args.append(mask2d)
        out_specs = [pl.BlockSpec((NBLK, DD), lambda i: (i, 0))] * 2
        out_shape = [jax.ShapeDtypeStruct((NN, DD), jnp.float32)] * 2
    else:
        out_specs = pl.BlockSpec((NBLK, DD), lambda i: (i, 0))
        out_shape = jax.ShapeDtypeStruct((NN, DD), jnp.float32)
    return pl.pallas_call(
        functools.partial(_mlp_body, relu_out, emit_masked),
        grid=(NN // NBLK,),
        in_specs=in_specs,
        out_specs=out_specs,
        out_shape=out_shape,
    )(*args)


def _add_relu_body(h_ref, p_ref0, p_ref1, oh_ref, og_ref):
    t = h_ref[...] + p_ref0[0] + p_ref1[0]
    oh_ref[...] = t
    og_ref[...] = jnp.maximum(t, 0.0)


def _add_relu(h, parts):
    return pl.pallas_call(
        _add_relu_body,
        grid=(NN // NBLK,),
        in_specs=[pl.BlockSpec((NBLK, DD), lambda i: (i, 0)),
                  pl.BlockSpec((1, NBLK, DD), lambda i: (0, i, 0)),
                  pl.BlockSpec((1, NBLK, DD), lambda i: (1, i, 0))],
        out_specs=[pl.BlockSpec((NBLK, DD), lambda i: (i, 0))] * 2,
        out_shape=[jax.ShapeDtypeStruct((NN, DD), jnp.float32)] * 2,
    )(h, parts, parts)


def kernel(x, edge_index, edge_attr, batch, expander_edge_index,
           expander_node_mask, num_nodes, emb, Wedge, bedge, eps_main,
           W1, b1, W2, b2, bn_g, bn_b, eps_r, rW1, rb1, rW2, rb2,
           rbn_g, rbn_b):
    src, dst = edge_index[0], edge_index[1]
    esrc, edst = expander_edge_index[0], expander_edge_index[1]
    mask2d = expander_node_mask.reshape(NN, 1)
    zeros = jnp.zeros((ROWS_PER_TILE, DD), jnp.float32)
    bns = 1.0 / jnp.sqrt(1.0 + 1e-5)

    E = src.shape[0]
    EE = esrc.shape[0]
    sc_main = _make_sc_pass(E, True)
    sc_plain = _make_sc_pass(EE, False)

    h = _h0(mask2d, emb)
    for l in range(3):
        e_emb = _edge_matmul(edge_attr, Wedge[l], bedge[l])
        part = sc_main(h, src, dst, e_emb, zeros)
        W2f = W2[l] * (bn_g[l] * bns)[None, :]
        b2f = b2[l] * (bn_g[l] * bns) + bn_b[l]
        if l < 2:
            _, hm = _node_mlp(h, part, eps_main[l], W1[l], b1[l], W2f, b2f,
                              relu_out=True, mask2d=mask2d)
            part2 = sc_plain(hm, esrc, edst, zeros)
            h_after, g = _add_relu(hm, part2)
            part3 = sc_plain(g, edst, esrc, zeros)
            rW2f = rW2[l] * (rbn_g[l] * bns)[None, :]
            rb2f = rb2[l] * (rbn_g[l] * bns) + rbn_b[l]
            h = _node_mlp(h_after, part3, eps_r[l], rW1[l], rb1[l],
                          rW2f, rb2f, relu_out=True)
        else:
            h = _node_mlp(h, part, eps_main[l], W1[l], b1[l], W2f, b2f,
                          relu_out=False)
    return h
